# baseline (speedup 1.0000x reference)
import functools

import jax
import jax.numpy as jnp
from jax import lax
from jax.experimental import pallas as pl
from jax.experimental.pallas import tpu as pltpu
from jax.experimental.pallas import tpu_sc as plsc

EMBED_DIM = 64
HALF = 100
NBUF = 4
PREF = 3


@functools.cache
def _build(batch, hist):
    info = plsc.get_sparse_core_info()
    nc, ns = info.num_cores, info.num_subcores
    nw = nc * ns
    nb_w = batch // nw
    nidx_w = batch * hist // (nw * HALF)
    assert hist == 2 * HALF
    assert nb_w * nw == batch and nb_w % NBUF == 0 and nb_w // NBUF >= 3

    mesh = plsc.VectorSubcoreMesh(core_axis_name="c", subcore_axis_name="s")

    @functools.partial(
        pl.kernel,
        mesh=mesh,
        out_type=jax.ShapeDtypeStruct((batch, hist, EMBED_DIM), jnp.float32),
        compiler_params=pltpu.CompilerParams(use_tc_tiling_on_sc=False),
        scratch_types=[
            pltpu.VMEM((batch * hist // (nw * HALF), HALF), jnp.int32),
            [pltpu.VMEM((hist, EMBED_DIM), jnp.float32)] * NBUF,
            [pltpu.SemaphoreType.DMA] * NBUF,
            [pltpu.SemaphoreType.DMA] * NBUF,
        ],
    )
    def gather_kernel(table_hbm, idx_hbm, out_hbm, idx_v, bufs, sg, sw):
        wid = lax.axis_index("s") * nc + lax.axis_index("c")
        b_base = wid * nb_w

        pltpu.sync_copy(idx_hbm.at[pl.ds(wid * nidx_w, nidx_w)], idx_v)

        def fire_g(i, b):
            pltpu.async_copy(
                table_hbm.at[idx_v.at[2 * i]], bufs[b].at[pl.ds(0, HALF)], sg[b]
            )
            pltpu.async_copy(
                table_hbm.at[idx_v.at[2 * i + 1]],
                bufs[b].at[pl.ds(HALF, HALF)],
                sg[b],
            )

        def drain_g(b):
            for _ in range(2):
                pltpu.make_async_copy(
                    table_hbm.at[idx_v.at[0]], bufs[b].at[pl.ds(0, HALF)], sg[b]
                ).wait()

        def fire_w(i, b):
            pltpu.async_copy(bufs[b], out_hbm.at[b_base + i], sw[b])

        def drain_w(b):
            pltpu.make_async_copy(bufs[b], out_hbm.at[b_base], sw[b]).wait()

        def step(i, b, do_drain_w, do_pref):
            drain_g(b)
            fire_w(i, b)
            if do_pref:
                if do_drain_w:
                    drain_w((b + PREF) % NBUF)
                fire_g(i + PREF, (b + PREF) % NBUF)

        for b in range(PREF):
            fire_g(b, b)
        for i in range(NBUF):
            step(i, i, do_drain_w=(i >= 1), do_pref=True)

        def body(r, carry):
            for db in range(NBUF):
                i = r * NBUF + db
                step(i, db, do_drain_w=True, do_pref=True)
            return carry

        lax.fori_loop(1, nb_w // NBUF - 1, body, 0, unroll=False)

        for db in range(NBUF):
            i = nb_w - NBUF + db
            step(i, db, do_drain_w=True, do_pref=(i + PREF < nb_w))
        for b in range(NBUF):
            drain_w(b)

    return gather_kernel


def kernel(indices, table):
    batch, hist = indices.shape
    idx2d = indices.astype(jnp.int32).reshape(batch * hist // HALF, HALF)
    return _build(batch, hist)(table, idx2d)
